# baseline (speedup 1.0000x reference)
import jax
import jax.numpy as jnp
from jax import lax
from jax.experimental import pallas as pl
from jax.experimental.pallas import tpu as pltpu
from jax.experimental.pallas import tpu_sc as plsc

B = 4096
T = 200
L = 50
LH = 25
D = 64

NW = 32
NL = B // NW
NG = NL // 16
DC = 2
NDCH = D // DC
LANES = 16


def _body(pt_hbm, it_hbm, ot_hbm, idx_v, slab_v, out_v, gsem, ssem):
    wid = lax.axis_index("s") * 2 + lax.axis_index("c")
    b0 = wid * NL

    def fire(c, s):
        pltpu.async_copy(
            pt_hbm.at[:, pl.ds(c * DC, DC), pl.ds(b0, NL)],
            slab_v.at[s], gsem.at[s])

    fire(0, 0)
    fire(1, 1)
    pltpu.sync_copy(it_hbm.at[:, pl.ds(b0, NL)], idx_v)

    iota = lax.iota(jnp.int32, LANES)
    lanes = [jnp.full((LANES,), g * LANES, jnp.int32) + iota for g in range(NG)]
    zero = jnp.zeros((LANES,), jnp.int32)
    tstride = jnp.full((LANES,), DC * NL, jnp.int32)
    dstride = jnp.full((LANES,), NL, jnp.int32)

    @plsc.parallel_loop(0, L, unroll=2)
    def _pre(l):
        for g in range(NG):
            sl = pl.ds(g * LANES, LANES)
            idx_v[l, sl] = idx_v[l, sl] * tstride + lanes[g]

    def chunk(c, s):
        pltpu.make_async_copy(
            pt_hbm.at[:, pl.ds(0, DC), pl.ds(b0, NL)],
            slab_v.at[s], gsem.at[s]).wait()

        for h in range(2):
            @pl.when(c > 0)
            def _wait_out_free():
                pltpu.make_async_copy(
                    out_v.at[h],
                    ot_hbm.at[pl.ds(h * LH, LH), pl.ds(0, DC), pl.ds(b0, NL)],
                    ssem.at[h]).wait()

            @plsc.parallel_loop(0, LH, unroll=4)
            def per_l(ll):
                for g in range(NG):
                    sl = pl.ds(g * LANES, LANES)
                    sidx = idx_v[ll + h * LH, sl]
                    for d in range(DC):
                        v = plsc.load_gather(
                            slab_v.at[s],
                            [zero, zero, sidx if d == 0 else sidx + d * dstride])
                        out_v[h, ll, d, sl] = v
            pltpu.async_copy(
                out_v.at[h],
                ot_hbm.at[pl.ds(h * LH, LH), pl.ds(c * DC, DC), pl.ds(b0, NL)],
                ssem.at[h])

        @pl.when(c + 2 < NDCH)
        def _refill():
            fire(c + 2, s)

    def group(g2, _):
        chunk(g2 * 2, 0)
        chunk(g2 * 2 + 1, 1)
        return _

    lax.fori_loop(0, NDCH // 2, group, None)
    for h in range(2):
        pltpu.make_async_copy(
            out_v.at[h],
            ot_hbm.at[pl.ds(h * LH, LH), pl.ds(0, DC), pl.ds(b0, NL)],
            ssem.at[h]).wait()


def kernel(params, indices):
    pt = params.transpose(1, 2, 0)
    it = indices.astype(jnp.int32).T

    mesh = plsc.VectorSubcoreMesh(core_axis_name="c", subcore_axis_name="s")
    k = pl.kernel(
        _body,
        mesh=mesh,
        out_type=jax.ShapeDtypeStruct((L, D, B), jnp.float32),
        scratch_types=[
            pltpu.VMEM((L, NL), jnp.int32),
            pltpu.VMEM((2, T, DC, NL), jnp.float32),
            pltpu.VMEM((2, LH, DC, NL), jnp.float32),
            pltpu.SemaphoreType.DMA((2,)),
            pltpu.SemaphoreType.DMA((2,)),
        ],
        compiler_params=pltpu.CompilerParams(use_tc_tiling_on_sc=True,
                                             needs_layout_passes=False),
    )
    ot = k(pt, it)
    return ot.transpose(2, 0, 1)


# speedup vs baseline: 1.0453x; 1.0453x over previous
import jax
import jax.numpy as jnp
from jax import lax
from jax.experimental import pallas as pl
from jax.experimental.pallas import tpu as pltpu
from jax.experimental.pallas import tpu_sc as plsc

B = 4096
T = 200
L = 50
LH = 25
D = 64

NW = 32
NL = B // NW
NG = NL // 16
DC = 2
NDCH = D // DC
LANES = 16


def _body(pt_hbm, it_hbm, ot_hbm, idx_v, slab_v, out_v, gsem, ssem):
    wid = lax.axis_index("s") * 2 + lax.axis_index("c")
    b0 = wid * NL

    def fire(c, s):
        pltpu.async_copy(
            pt_hbm.at[:, pl.ds(c * DC, DC), pl.ds(b0, NL)],
            slab_v.at[s], gsem.at[s])

    fire(0, 0)
    fire(1, 1)
    pltpu.sync_copy(it_hbm.at[:, pl.ds(b0, NL)], idx_v)

    iota = lax.iota(jnp.int32, LANES)
    lanes = [jnp.full((LANES,), g * LANES, jnp.int32) + iota for g in range(NG)]
    zero = jnp.zeros((LANES,), jnp.int32)
    tstride = jnp.full((LANES,), DC * NL, jnp.int32)
    dstride = jnp.full((LANES,), NL, jnp.int32)

    @plsc.parallel_loop(0, L, unroll=2)
    def _pre(l):
        for g in range(NG):
            sl = pl.ds(g * LANES, LANES)
            idx_v[l, sl] = idx_v[l, sl] * tstride + lanes[g]

    def chunk(c, s):
        pltpu.make_async_copy(
            pt_hbm.at[:, pl.ds(0, DC), pl.ds(b0, NL)],
            slab_v.at[s], gsem.at[s]).wait()

        @pl.when(c > 0)
        def _wait_out_free():
            pltpu.make_async_copy(
                out_v, ot_hbm.at[:, pl.ds(0, DC), pl.ds(b0, NL)],
                ssem).wait()

        @plsc.parallel_loop(0, L, unroll=4)
        def per_l(l):
            for g in range(NG):
                sl = pl.ds(g * LANES, LANES)
                sidx = idx_v[l, sl]
                for d in range(DC):
                    v = plsc.load_gather(
                        slab_v.at[s],
                        [zero, zero, sidx if d == 0 else sidx + d * dstride])
                    out_v[l, d, sl] = v
        pltpu.async_copy(out_v, ot_hbm.at[:, pl.ds(c * DC, DC), pl.ds(b0, NL)],
                         ssem)

        @pl.when(c + 2 < NDCH)
        def _refill():
            fire(c + 2, s)

    def group(g2, _):
        chunk(g2 * 2, 0)
        chunk(g2 * 2 + 1, 1)
        return _

    lax.fori_loop(0, NDCH // 2, group, None)
    pltpu.make_async_copy(
        out_v, ot_hbm.at[:, pl.ds(0, DC), pl.ds(b0, NL)], ssem).wait()


def kernel(params, indices):
    pt = params.transpose(1, 2, 0)
    it = indices.astype(jnp.int32).T

    mesh = plsc.VectorSubcoreMesh(core_axis_name="c", subcore_axis_name="s")
    k = pl.kernel(
        _body,
        mesh=mesh,
        out_type=jax.ShapeDtypeStruct((L, D, B), jnp.float32),
        scratch_types=[
            pltpu.VMEM((L, NL), jnp.int32),
            pltpu.VMEM((2, T, DC, NL), jnp.float32),
            pltpu.VMEM((L, DC, NL), jnp.float32),
            pltpu.SemaphoreType.DMA((2,)),
            pltpu.SemaphoreType.DMA,
        ],
        compiler_params=pltpu.CompilerParams(use_tc_tiling_on_sc=True,
                                             needs_layout_passes=False),
    )
    ot = k(pt, it)
    return ot.transpose(2, 0, 1)
